# baseline (speedup 1.0000x reference)
import functools

import jax
import jax.numpy as jnp
from jax import lax
from jax.experimental import pallas as pl
from jax.experimental.pallas import tpu as pltpu
from jax.experimental.pallas import tpu_sc as plsc

B = 1024
E = 64
V = 100000

_NC = 2
_NS = 16
_NW = _NC * _NS
_EPW = E // _NW


@functools.cache
def _make_sc_gather_t():
    mesh = plsc.VectorSubcoreMesh(core_axis_name="c", subcore_axis_name="s")

    @functools.partial(
        pl.kernel,
        mesh=mesh,
        out_type=jax.ShapeDtypeStruct((E, B), jnp.float32),
        scratch_types=[
            pltpu.VMEM((B,), jnp.int32),
            pltpu.VMEM((V,), jnp.float32),
            pltpu.VMEM((B,), jnp.float32),
        ],
        compiler_params=pltpu.CompilerParams(
            use_tc_tiling_on_sc=True, needs_layout_passes=False),
    )
    def _sc_gather_t(embT_hbm, idx_hbm, out_hbm, idx_v, row_v, hrow_v):
        wid = lax.axis_index("s") * _NC + lax.axis_index("c")
        pltpu.sync_copy(idx_hbm, idx_v)
        for r in range(_EPW):
            e = wid * _EPW + r
            pltpu.sync_copy(embT_hbm.at[e], row_v)

            def body(j, carry):
                idx = idx_v[pl.ds(j * 16, 16)]
                hrow_v[pl.ds(j * 16, 16)] = plsc.load_gather(row_v, [idx])
                return carry

            lax.fori_loop(0, B // 16, body, 0)
            pltpu.sync_copy(hrow_v, out_hbm.at[e])

    return _sc_gather_t


_VB = 4096
_NG = (V + _VB - 1) // _VB
_VPAD = _NG * _VB


def _proj_body(wt_ref, h_ref, b_ref, out_ref):
    acc = lax.dot_general(
        wt_ref[...], h_ref[...],
        dimension_numbers=(((0,), (0,)), ((), ())),
        preferred_element_type=jnp.float32,
    )
    ones_row = jnp.ones((1, B), dtype=jnp.float32)
    bias = lax.dot_general(
        b_ref[0], ones_row,
        dimension_numbers=(((0,), (0,)), ((), ())),
        preferred_element_type=jnp.float32,
    )
    out_ref[...] = acc + bias


def _tc_project(hT, Wt, b3):
    outT = pl.pallas_call(
        _proj_body,
        grid=(_NG,),
        in_specs=[
            pl.BlockSpec((E, _VB), lambda i: (0, i)),
            pl.BlockSpec((E, B), lambda i: (0, 0)),
            pl.BlockSpec((1, 1, _VB), lambda i: (i, 0, 0)),
        ],
        out_specs=pl.BlockSpec((_VB, B), lambda i: (i, 0)),
        out_shape=jax.ShapeDtypeStruct((V, B), jnp.float32),
    )(Wt, hT, b3)
    return outT.T


def kernel(x, emb, W, b):
    hT = _make_sc_gather_t()(emb.T, x.astype(jnp.int32))
    Wt = W.T
    b3 = jnp.pad(b, (0, _VPAD - V)).reshape(_NG, 1, _VB)
    return _tc_project(hT, Wt, b3)
